# Initial kernel scaffold; baseline (speedup 1.0000x reference)
import jax
import jax.numpy as jnp
from jax.experimental import pallas as pl


def kernel(obs, edge_index, W1, b1, W2, b2):
    raise NotImplementedError("write your pallas kernel here")


# speedup vs baseline: 36.2263x; 36.2263x over previous
import functools

import jax
import jax.numpy as jnp
from jax import lax
from jax.experimental import pallas as pl
from jax.experimental.pallas import tpu as pltpu
from jax.experimental.pallas import tpu_sc as plsc

NC = 2
NS = 16
NW = NC * NS


def _fill(ref, value, n):
    def body(i, _):
        ref[pl.ds(pl.multiple_of(i * 16, 16), 16)] = jnp.full(
            (16,), value, jnp.float32)
        return 0
    lax.fori_loop(0, n // 16, body, 0)


def _fill2d(ref, value, rows, cols):
    def body(i, _):
        r = i // (cols // 16)
        q = i % (cols // 16)
        ref[r, pl.ds(pl.multiple_of(q * 16, 16), 16)] = jnp.full(
            (16,), value, jnp.float32)
        return 0
    lax.fori_loop(0, rows * (cols // 16), body, 0)


def _make_sc_kernels(n_pad, ec, fh):
    nslice = n_pad // NS
    mesh = plsc.VectorSubcoreMesh(core_axis_name="c", subcore_axis_name="s")
    sc_params = pltpu.CompilerParams(use_tc_tiling_on_sc=False)

    @functools.partial(
        pl.kernel,
        out_type=jax.ShapeDtypeStruct((NC * n_pad,), jnp.float32),
        mesh=mesh,
        compiler_params=sc_params,
        scratch_types=[
            pltpu.VMEM((ec, 128), jnp.int32),
            pltpu.VMEM((128,), jnp.float32),
            pltpu.VMEM((nslice,), jnp.float32),
            pltpu.VMEM_SHARED((n_pad,), jnp.float32),
            pltpu.SemaphoreType.DMA,
        ],
    )
    def deg_kernel(dst_hbm, out_hbm, dst_v, ones_v, buf_v, acc_sh, sem):
        c = lax.axis_index("c")
        s = lax.axis_index("s")
        gid = c * NS + s
        _fill(ones_v, 1.0, 128)
        _fill(buf_v, 0.0, nslice)
        pltpu.sync_copy(buf_v, acc_sh.at[pl.ds(s * nslice, nslice)])
        plsc.subcore_barrier()
        pltpu.sync_copy(dst_hbm.at[pl.ds(gid * ec, ec)], dst_v)

        def body(i, _):
            pltpu.sync_copy(ones_v, acc_sh.at[dst_v.at[i]], add=True)
            return 0
        lax.fori_loop(0, ec, body, 0)
        plsc.subcore_barrier()
        pltpu.sync_copy(acc_sh.at[pl.ds(s * nslice, nslice)], buf_v)
        pltpu.sync_copy(buf_v, out_hbm.at[pl.ds(c * n_pad + s * nslice, nslice)])

    @functools.partial(
        pl.kernel,
        out_type=jax.ShapeDtypeStruct((NC * n_pad, fh), jnp.float32),
        mesh=mesh,
        compiler_params=sc_params,
        scratch_types=[
            pltpu.VMEM((ec, 128), jnp.int32),
            pltpu.VMEM((ec, 128), jnp.int32),
            pltpu.VMEM((128, fh), jnp.float32),
            pltpu.VMEM_SHARED((n_pad, fh), jnp.float32),
            pltpu.SemaphoreType.DMA,
        ],
    )
    def agg1_kernel(u_hbm, src_hbm, dst_hbm, out_hbm,
                    src_v, dst_v, rows_v, acc_sh, sem):
        c = lax.axis_index("c")
        s = lax.axis_index("s")
        gid = c * NS + s
        _fill2d(rows_v, 0.0, 128, fh)

        def zero(k, _):
            pltpu.sync_copy(rows_v, acc_sh.at[pl.ds(s * nslice + k * 128, 128)])
            return 0
        lax.fori_loop(0, nslice // 128, zero, 0)
        plsc.subcore_barrier()
        pltpu.sync_copy(src_hbm.at[pl.ds(gid * ec, ec)], src_v)
        pltpu.sync_copy(dst_hbm.at[pl.ds(gid * ec, ec)], dst_v)

        def body(i, _):
            pltpu.async_copy(u_hbm.at[src_v.at[i]], rows_v, sem).wait()
            pltpu.sync_copy(rows_v, acc_sh.at[dst_v.at[i]], add=True)
            return 0
        lax.fori_loop(0, ec, body, 0)
        plsc.subcore_barrier()

        def wb(k, _):
            pltpu.sync_copy(acc_sh.at[pl.ds(s * nslice + k * 128, 128)], rows_v)
            pltpu.sync_copy(
                rows_v,
                out_hbm.at[pl.ds(c * n_pad + s * nslice + k * 128, 128)])
            return 0
        lax.fori_loop(0, nslice // 128, wb, 0)

    @functools.partial(
        pl.kernel,
        out_type=jax.ShapeDtypeStruct((NC * n_pad,), jnp.float32),
        mesh=mesh,
        compiler_params=sc_params,
        scratch_types=[
            pltpu.VMEM((ec, 128), jnp.int32),
            pltpu.VMEM((ec, 128), jnp.int32),
            pltpu.VMEM((128,), jnp.float32),
            pltpu.VMEM((nslice,), jnp.float32),
            pltpu.VMEM_SHARED((n_pad,), jnp.float32),
            pltpu.SemaphoreType.DMA,
        ],
    )
    def agg2_kernel(t_hbm, src_hbm, dst_hbm, out_hbm,
                    src_v, dst_v, vals_v, buf_v, acc_sh, sem):
        c = lax.axis_index("c")
        s = lax.axis_index("s")
        gid = c * NS + s
        _fill(buf_v, 0.0, nslice)
        pltpu.sync_copy(buf_v, acc_sh.at[pl.ds(s * nslice, nslice)])
        plsc.subcore_barrier()
        pltpu.sync_copy(src_hbm.at[pl.ds(gid * ec, ec)], src_v)
        pltpu.sync_copy(dst_hbm.at[pl.ds(gid * ec, ec)], dst_v)

        def body(i, _):
            pltpu.async_copy(t_hbm.at[src_v.at[i]], vals_v, sem).wait()
            pltpu.sync_copy(vals_v, acc_sh.at[dst_v.at[i]], add=True)
            return 0
        lax.fori_loop(0, ec, body, 0)
        plsc.subcore_barrier()
        pltpu.sync_copy(acc_sh.at[pl.ds(s * nslice, nslice)], buf_v)
        pltpu.sync_copy(buf_v, out_hbm.at[pl.ds(c * n_pad + s * nslice, nslice)])

    return deg_kernel, agg1_kernel, agg2_kernel


def kernel(obs, edge_index, W1, b1, W2, b2):
    n, fin = obs.shape
    fh = W1.shape[1]
    e = edge_index.shape[1]

    n_pad = ((n + NS * 128 - 1) // (NS * 128)) * (NS * 128)
    e_pad = ((e + NW * 1024 - 1) // (NW * 1024)) * (NW * 1024)
    ec = e_pad // NW // 128

    src = edge_index[0].astype(jnp.int32)
    dst = edge_index[1].astype(jnp.int32)
    src2d = jnp.concatenate(
        [src, jnp.zeros((e_pad - e,), jnp.int32)]).reshape(e_pad // 128, 128)
    dst2d = jnp.concatenate(
        [dst, jnp.full((e_pad - e,), n, jnp.int32)]).reshape(e_pad // 128, 128)
    obs_p = jnp.pad(obs, ((0, n_pad - n), (0, 0)))

    deg_kernel, agg1_kernel, agg2_kernel = _make_sc_kernels(n_pad, ec, fh)

    indeg = deg_kernel(dst2d).reshape(NC, n_pad, 1)

    def _prep(obs_ref, w1_ref, indeg_ref, u_ref, d_ref):
        deg = indeg_ref[0] + indeg_ref[1] + 1.0
        d = lax.rsqrt(deg)
        xw = jnp.dot(obs_ref[...], w1_ref[...],
                     preferred_element_type=jnp.float32,
                     precision=lax.Precision.HIGHEST)
        u_ref[...] = xw * d
        d_ref[...] = d

    u, dcol = pl.pallas_call(
        _prep,
        out_shape=(jax.ShapeDtypeStruct((n_pad, fh), jnp.float32),
                   jax.ShapeDtypeStruct((n_pad, 1), jnp.float32)),
    )(obs_p, W1, indeg)

    acc1 = agg1_kernel(u, src2d, dst2d).reshape(NC, n_pad, fh)

    def _mid(acc_ref, u_ref, d_ref, b1_ref, w2_ref, t_ref):
        acc = acc_ref[0] + acc_ref[1]
        d = d_ref[...]
        h1 = jnp.maximum((acc + u_ref[...]) * d + b1_ref[...], 0.0)
        t_ref[...] = jnp.dot(h1, w2_ref[...],
                             preferred_element_type=jnp.float32,
                             precision=lax.Precision.HIGHEST) * d

    t = pl.pallas_call(
        _mid,
        out_shape=jax.ShapeDtypeStruct((n_pad, 1), jnp.float32),
    )(acc1, u, dcol, b1.reshape(1, fh), W2)

    acc2 = agg2_kernel(t.reshape(n_pad), src2d, dst2d)

    rows128 = n_pad // 128

    def _fin(acc2_ref, t_ref, d_ref, b2_ref, h2_ref):
        acc = acc2_ref[0] + acc2_ref[1]
        h2_ref[...] = jnp.maximum(
            (acc + t_ref[...]) * d_ref[...] + b2_ref[...], 0.0)

    h2 = pl.pallas_call(
        _fin,
        out_shape=jax.ShapeDtypeStruct((rows128, 128), jnp.float32),
    )(acc2.reshape(NC, rows128, 128), t.reshape(rows128, 128),
      dcol.reshape(rows128, 128), b2.reshape(1, 1))

    return h2.reshape(n_pad)[:n].reshape(-1, 15)[:, 3:].reshape(-1)


# speedup vs baseline: 49.0434x; 1.3538x over previous
import functools

import jax
import jax.numpy as jnp
from jax import lax
from jax.experimental import pallas as pl
from jax.experimental.pallas import tpu as pltpu
from jax.experimental.pallas import tpu_sc as plsc

NC = 2
NS = 16
NW = NC * NS


def _fill(ref, value, n):
    def body(i, _):
        ref[pl.ds(pl.multiple_of(i * 16, 16), 16)] = jnp.full(
            (16,), value, jnp.float32)
        return 0
    lax.fori_loop(0, n // 16, body, 0)


def _fill2d(ref, value, rows, cols):
    def body(i, _):
        r = i // (cols // 16)
        q = i % (cols // 16)
        ref[r, pl.ds(pl.multiple_of(q * 16, 16), 16)] = jnp.full(
            (16,), value, jnp.float32)
        return 0
    lax.fori_loop(0, rows * (cols // 16), body, 0)


def _make_sc_kernels(n_pad, ec, fh):
    nslice = n_pad // NS
    mesh = plsc.VectorSubcoreMesh(core_axis_name="c", subcore_axis_name="s")
    sc_params = pltpu.CompilerParams(
        use_tc_tiling_on_sc=False, needs_layout_passes=False)

    @functools.partial(
        pl.kernel,
        out_type=jax.ShapeDtypeStruct((NC * n_pad,), jnp.float32),
        mesh=mesh,
        compiler_params=sc_params,
        scratch_types=[
            pltpu.VMEM((ec, 128), jnp.int32),
            pltpu.VMEM((128,), jnp.float32),
            pltpu.VMEM((nslice,), jnp.float32),
            pltpu.VMEM_SHARED((n_pad,), jnp.float32),
            pltpu.SemaphoreType.DMA,
        ],
    )
    def deg_kernel(dst_hbm, out_hbm, dst_v, ones_v, buf_v, acc_sh, sem):
        c = lax.axis_index("c")
        s = lax.axis_index("s")
        gid = c * NS + s
        _fill(ones_v, 1.0, 128)
        _fill(buf_v, 0.0, nslice)
        pltpu.sync_copy(buf_v, acc_sh.at[pl.ds(s * nslice, nslice)])
        plsc.subcore_barrier()
        pltpu.sync_copy(dst_hbm.at[pl.ds(gid * ec, ec)], dst_v)

        def fire(i, _):
            pltpu.async_copy(ones_v, acc_sh.at[dst_v.at[i]], sem, add=True)
            return 0
        lax.fori_loop(0, ec, fire, 0)

        def drain(i, _):
            pltpu.make_async_copy(ones_v, acc_sh.at[dst_v.at[i]], sem).wait()
            return 0
        lax.fori_loop(0, ec, drain, 0)
        plsc.subcore_barrier()
        pltpu.sync_copy(acc_sh.at[pl.ds(s * nslice, nslice)], buf_v)
        pltpu.sync_copy(buf_v, out_hbm.at[pl.ds(c * n_pad + s * nslice, nslice)])

    nb = 8
    assert ec % nb == 0

    @functools.partial(
        pl.kernel,
        out_type=jax.ShapeDtypeStruct((NC * n_pad, fh), jnp.float32),
        mesh=mesh,
        compiler_params=sc_params,
        scratch_types=[
            pltpu.VMEM((ec, 128), jnp.int32),
            pltpu.VMEM((ec, 128), jnp.int32),
            pltpu.VMEM_SHARED((n_pad, fh), jnp.float32),
        ] + [pltpu.VMEM((128, fh), jnp.float32) for _ in range(nb)]
          + [pltpu.SemaphoreType.DMA for _ in range(2 * nb)],
    )
    def agg1_kernel(u_hbm, src_hbm, dst_hbm, out_hbm,
                    src_v, dst_v, acc_sh, *rest):
        rows = rest[:nb]
        gsem = rest[nb:2 * nb]
        ssem = rest[2 * nb:3 * nb]
        c = lax.axis_index("c")
        s = lax.axis_index("s")
        gid = c * NS + s
        _fill2d(rows[0], 0.0, 128, fh)

        def zero(k, _):
            pltpu.sync_copy(rows[0], acc_sh.at[pl.ds(s * nslice + k * 128, 128)])
            return 0
        lax.fori_loop(0, nslice // 128, zero, 0)
        plsc.subcore_barrier()
        pltpu.sync_copy(src_hbm.at[pl.ds(gid * ec, ec)], src_v)
        pltpu.sync_copy(dst_hbm.at[pl.ds(gid * ec, ec)], dst_v)

        for b in range(nb):
            pltpu.async_copy(u_hbm.at[src_v.at[b]], rows[b], gsem[b])

        def round_body(r, _):
            for b in range(nb):
                j = r * nb + b
                pltpu.make_async_copy(
                    u_hbm.at[src_v.at[j]], rows[b], gsem[b]).wait()
                pltpu.async_copy(
                    rows[b], acc_sh.at[dst_v.at[j]], ssem[b], add=True)
            for b in range(nb):
                j = r * nb + b

                @pl.when(j + nb < ec)
                def _(b=b, j=j):
                    pltpu.make_async_copy(
                        rows[b], acc_sh.at[dst_v.at[j]], ssem[b]).wait()
                    pltpu.async_copy(
                        u_hbm.at[src_v.at[j + nb]], rows[b], gsem[b])
            return 0
        lax.fori_loop(0, ec // nb, round_body, 0)
        for b in range(nb):
            pltpu.make_async_copy(
                rows[b], acc_sh.at[dst_v.at[ec - nb + b]], ssem[b]).wait()
        plsc.subcore_barrier()

        def wb(k, _):
            pltpu.sync_copy(acc_sh.at[pl.ds(s * nslice + k * 128, 128)], rows[0])
            pltpu.sync_copy(
                rows[0],
                out_hbm.at[pl.ds(c * n_pad + s * nslice + k * 128, 128)])
            return 0
        lax.fori_loop(0, nslice // 128, wb, 0)

    @functools.partial(
        pl.kernel,
        out_type=jax.ShapeDtypeStruct((NC * n_pad,), jnp.float32),
        mesh=mesh,
        compiler_params=sc_params,
        scratch_types=[
            pltpu.VMEM((ec, 128), jnp.int32),
            pltpu.VMEM((ec, 128), jnp.int32),
            pltpu.VMEM((ec, 128), jnp.float32),
            pltpu.VMEM((n_pad,), jnp.float32),
            pltpu.VMEM((nslice,), jnp.float32),
            pltpu.VMEM_SHARED((n_pad,), jnp.float32),
            pltpu.SemaphoreType.DMA,
        ],
    )
    def agg2_kernel(t_hbm, src_hbm, dst_hbm, out_hbm,
                    src_v, dst_v, vals_v, t_local, buf_v, acc_sh, sem):
        c = lax.axis_index("c")
        s = lax.axis_index("s")
        gid = c * NS + s
        _fill(buf_v, 0.0, nslice)
        pltpu.sync_copy(buf_v, acc_sh.at[pl.ds(s * nslice, nslice)])
        pltpu.sync_copy(t_hbm, t_local)
        pltpu.sync_copy(src_hbm.at[pl.ds(gid * ec, ec)], src_v)
        pltpu.sync_copy(dst_hbm.at[pl.ds(gid * ec, ec)], dst_v)

        def gather_body(i, _):
            j = i // 8
            q = pl.multiple_of((i % 8) * 16, 16)
            idx = src_v[j, pl.ds(q, 16)]
            vals_v[j, pl.ds(q, 16)] = plsc.load_gather(t_local, [idx])
            return 0
        lax.fori_loop(0, ec * 8, gather_body, 0)
        plsc.subcore_barrier()

        def fire(i, _):
            pltpu.async_copy(vals_v.at[i], acc_sh.at[dst_v.at[i]], sem, add=True)
            return 0
        lax.fori_loop(0, ec, fire, 0)

        def drain(i, _):
            pltpu.make_async_copy(vals_v.at[i], acc_sh.at[dst_v.at[i]], sem).wait()
            return 0
        lax.fori_loop(0, ec, drain, 0)
        plsc.subcore_barrier()
        pltpu.sync_copy(acc_sh.at[pl.ds(s * nslice, nslice)], buf_v)
        pltpu.sync_copy(buf_v, out_hbm.at[pl.ds(c * n_pad + s * nslice, nslice)])

    return deg_kernel, agg1_kernel, agg2_kernel


def kernel(obs, edge_index, W1, b1, W2, b2):
    n, fin = obs.shape
    fh = W1.shape[1]
    e = edge_index.shape[1]

    n_pad = ((n + NS * 128 - 1) // (NS * 128)) * (NS * 128)
    e_pad = ((e + NW * 1024 - 1) // (NW * 1024)) * (NW * 1024)
    ec = e_pad // NW // 128

    src = edge_index[0].astype(jnp.int32)
    dst = edge_index[1].astype(jnp.int32)
    src2d = jnp.concatenate(
        [src, jnp.zeros((e_pad - e,), jnp.int32)]).reshape(e_pad // 128, 128)
    dst2d = jnp.concatenate(
        [dst, jnp.full((e_pad - e,), n, jnp.int32)]).reshape(e_pad // 128, 128)
    obs_p = jnp.pad(obs, ((0, n_pad - n), (0, 0)))

    deg_kernel, agg1_kernel, agg2_kernel = _make_sc_kernels(n_pad, ec, fh)

    indeg = deg_kernel(dst2d).reshape(NC, n_pad, 1)

    def _prep(obs_ref, w1_ref, indeg_ref, u_ref, d_ref):
        deg = indeg_ref[0] + indeg_ref[1] + 1.0
        d = lax.rsqrt(deg)
        xw = jnp.dot(obs_ref[...], w1_ref[...],
                     preferred_element_type=jnp.float32,
                     precision=lax.Precision.HIGHEST)
        u_ref[...] = xw * d
        d_ref[...] = d

    u, dcol = pl.pallas_call(
        _prep,
        out_shape=(jax.ShapeDtypeStruct((n_pad, fh), jnp.float32),
                   jax.ShapeDtypeStruct((n_pad, 1), jnp.float32)),
    )(obs_p, W1, indeg)

    acc1 = agg1_kernel(u, src2d, dst2d).reshape(NC, n_pad, fh)

    def _mid(acc_ref, u_ref, d_ref, b1_ref, w2_ref, t_ref):
        acc = acc_ref[0] + acc_ref[1]
        d = d_ref[...]
        h1 = jnp.maximum((acc + u_ref[...]) * d + b1_ref[...], 0.0)
        t_ref[...] = jnp.dot(h1, w2_ref[...],
                             preferred_element_type=jnp.float32,
                             precision=lax.Precision.HIGHEST) * d

    t = pl.pallas_call(
        _mid,
        out_shape=jax.ShapeDtypeStruct((n_pad, 1), jnp.float32),
    )(acc1, u, dcol, b1.reshape(1, fh), W2)

    acc2 = agg2_kernel(t.reshape(n_pad), src2d, dst2d)

    rows128 = n_pad // 128

    def _fin(acc2_ref, t_ref, d_ref, b2_ref, h2_ref):
        acc = acc2_ref[0] + acc2_ref[1]
        h2_ref[...] = jnp.maximum(
            (acc + t_ref[...]) * d_ref[...] + b2_ref[...], 0.0)

    h2 = pl.pallas_call(
        _fin,
        out_shape=jax.ShapeDtypeStruct((rows128, 128), jnp.float32),
    )(acc2.reshape(NC, rows128, 128), t.reshape(rows128, 128),
      dcol.reshape(rows128, 128), b2.reshape(1, 1))

    return h2.reshape(n_pad)[:n].reshape(-1, 15)[:, 3:].reshape(-1)


# speedup vs baseline: 83.5853x; 1.7043x over previous
import functools

import jax
import jax.numpy as jnp
from jax import lax
from jax.experimental import pallas as pl
from jax.experimental.pallas import tpu as pltpu
from jax.experimental.pallas import tpu_sc as plsc

NC = 2
NS = 16
NW = NC * NS


def _fill(ref, value, n):
    def body(i, _):
        ref[pl.ds(pl.multiple_of(i * 16, 16), 16)] = jnp.full(
            (16,), value, jnp.float32)
        return 0
    lax.fori_loop(0, n // 16, body, 0)


def _fill2d(ref, value, rows, cols):
    def body(i, _):
        r = i // (cols // 16)
        q = pl.multiple_of((i % (cols // 16)) * 16, 16)
        ref[r, pl.ds(q, 16)] = jnp.full((16,), value, jnp.float32)
        return 0
    lax.fori_loop(0, rows * (cols // 16), body, 0)


def _make_sc_kernels(n_pad, ec, fh):
    nslice = n_pad // NS
    mesh = plsc.VectorSubcoreMesh(core_axis_name="c", subcore_axis_name="s")
    sc_params = pltpu.CompilerParams(
        use_tc_tiling_on_sc=False, needs_layout_passes=False)
    sds = jax.ShapeDtypeStruct

    @functools.partial(
        pl.kernel,
        out_type=(sds((n_pad,), jnp.float32), sds((n_pad,), jnp.float32)),
        mesh=mesh,
        compiler_params=sc_params,
        scratch_types=[
            pltpu.VMEM((ec, 128), jnp.int32),
            pltpu.VMEM((128,), jnp.float32),
            pltpu.VMEM((nslice,), jnp.float32),
            pltpu.VMEM_SHARED((n_pad,), jnp.float32),
            pltpu.SemaphoreType.DMA,
        ],
    )
    def deg_kernel(dst_hbm, out_a, out_b, dst_v, ones_v, buf_v, acc_sh, sem):
        c = lax.axis_index("c")
        s = lax.axis_index("s")
        gid = c * NS + s
        _fill(ones_v, 1.0, 128)
        _fill(buf_v, 0.0, nslice)
        pltpu.sync_copy(buf_v, acc_sh.at[pl.ds(s * nslice, nslice)])
        plsc.subcore_barrier()
        pltpu.sync_copy(dst_hbm.at[pl.ds(gid * ec, ec)], dst_v)

        def fire(i, _):
            pltpu.async_copy(ones_v, acc_sh.at[dst_v.at[i]], sem, add=True)
            return 0
        lax.fori_loop(0, ec, fire, 0)

        def drain(i, _):
            pltpu.make_async_copy(ones_v, acc_sh.at[dst_v.at[i]], sem).wait()
            return 0
        lax.fori_loop(0, ec, drain, 0)
        plsc.subcore_barrier()
        pltpu.sync_copy(acc_sh.at[pl.ds(s * nslice, nslice)], buf_v)

        @pl.when(c == 0)
        def _():
            pltpu.sync_copy(buf_v, out_a.at[pl.ds(s * nslice, nslice)])

        @pl.when(c == 1)
        def _():
            pltpu.sync_copy(buf_v, out_b.at[pl.ds(s * nslice, nslice)])

    nb = 5
    assert ec % nb == 0

    @functools.partial(
        pl.kernel,
        out_type=(sds((n_pad, fh), jnp.float32), sds((n_pad, fh), jnp.float32)),
        mesh=mesh,
        compiler_params=sc_params,
        scratch_types=[
            pltpu.VMEM_SHARED((n_pad, fh), jnp.float32),
            pltpu.VMEM_SHARED((n_pad, fh), jnp.float32),
        ] + [pltpu.VMEM((128, fh), jnp.float32) for _ in range(nb)]
          + [pltpu.VMEM((1, 128), jnp.int32) for _ in range(2 * nb)]
          + [pltpu.SemaphoreType.DMA for _ in range(4 * nb)],
    )
    def agg1_kernel(u_hbm, src_hbm, dst_hbm, out_a, out_b,
                    u_sh, acc_sh, *rest):
        rows = rest[:nb]
        srcr = rest[nb:2 * nb]
        dstr = rest[2 * nb:3 * nb]
        gsem = rest[3 * nb:4 * nb]
        ssem = rest[4 * nb:5 * nb]
        srcsem = rest[5 * nb:6 * nb]
        dstsem = rest[6 * nb:7 * nb]
        c = lax.axis_index("c")
        s = lax.axis_index("s")
        gid = c * NS + s

        def src_load(b, j):
            pltpu.async_copy(src_hbm.at[pl.ds(gid * ec + j, 1)], srcr[b],
                             srcsem[b])

        def src_wait(b):
            pltpu.make_async_copy(src_hbm.at[pl.ds(gid * ec, 1)], srcr[b],
                                  srcsem[b]).wait()

        def dst_load(b, j):
            pltpu.async_copy(dst_hbm.at[pl.ds(gid * ec + j, 1)], dstr[b],
                             dstsem[b])

        def dst_wait(b):
            pltpu.make_async_copy(dst_hbm.at[pl.ds(gid * ec, 1)], dstr[b],
                                  dstsem[b]).wait()

        pltpu.sync_copy(u_hbm.at[pl.ds(s * nslice, nslice)],
                        u_sh.at[pl.ds(s * nslice, nslice)])
        _fill2d(rows[0], 0.0, 128, fh)

        def zero(k, _):
            pltpu.sync_copy(rows[0], acc_sh.at[pl.ds(s * nslice + k * 128, 128)])
            return 0
        lax.fori_loop(0, nslice // 128, zero, 0)
        plsc.subcore_barrier()

        for b in range(nb):
            src_load(b, b)
            dst_load(b, b)
        for b in range(nb):
            src_wait(b)
            pltpu.async_copy(u_sh.at[srcr[b].at[0]], rows[b], gsem[b])

        def round_body(r, _):
            for b in range(nb):
                j = r * nb + b
                pltpu.make_async_copy(
                    u_sh.at[srcr[b].at[0]], rows[b], gsem[b]).wait()

                @pl.when(j + nb < ec)
                def _(b=b, j=j):
                    src_load(b, j + nb)
                dst_wait(b)
                pltpu.async_copy(
                    rows[b], acc_sh.at[dstr[b].at[0]], ssem[b], add=True)
            for b in range(nb):
                j = r * nb + b

                @pl.when(j + nb < ec)
                def _(b=b, j=j):
                    pltpu.make_async_copy(
                        rows[b], acc_sh.at[dstr[b].at[0]], ssem[b]).wait()
                    dst_load(b, j + nb)
                    src_wait(b)
                    pltpu.async_copy(u_sh.at[srcr[b].at[0]], rows[b], gsem[b])
            return 0
        lax.fori_loop(0, ec // nb, round_body, 0)
        for b in range(nb):
            pltpu.make_async_copy(
                rows[b], acc_sh.at[dstr[b].at[0]], ssem[b]).wait()
        plsc.subcore_barrier()

        def wb(k, _):
            pltpu.sync_copy(acc_sh.at[pl.ds(s * nslice + k * 128, 128)], rows[0])

            @pl.when(c == 0)
            def _():
                pltpu.sync_copy(
                    rows[0], out_a.at[pl.ds(s * nslice + k * 128, 128)])

            @pl.when(c == 1)
            def _():
                pltpu.sync_copy(
                    rows[0], out_b.at[pl.ds(s * nslice + k * 128, 128)])
            return 0
        lax.fori_loop(0, nslice // 128, wb, 0)

    @functools.partial(
        pl.kernel,
        out_type=(sds((n_pad,), jnp.float32), sds((n_pad,), jnp.float32)),
        mesh=mesh,
        compiler_params=sc_params,
        scratch_types=[
            pltpu.VMEM((ec, 128), jnp.int32),
            pltpu.VMEM((ec, 128), jnp.int32),
            pltpu.VMEM((ec, 128), jnp.float32),
            pltpu.VMEM((n_pad,), jnp.float32),
            pltpu.VMEM((nslice,), jnp.float32),
            pltpu.VMEM_SHARED((n_pad,), jnp.float32),
            pltpu.VMEM_SHARED((n_pad,), jnp.float32),
            pltpu.SemaphoreType.DMA,
        ],
    )
    def agg2_kernel(t_hbm, src_hbm, dst_hbm, out_a, out_b,
                    src_v, dst_v, vals_v, t_local, buf_v, t_sh, acc_sh, sem):
        c = lax.axis_index("c")
        s = lax.axis_index("s")
        gid = c * NS + s
        _fill(buf_v, 0.0, nslice)
        pltpu.sync_copy(buf_v, acc_sh.at[pl.ds(s * nslice, nslice)])
        pltpu.sync_copy(t_hbm.at[pl.ds(s * nslice, nslice)],
                        t_sh.at[pl.ds(s * nslice, nslice)])
        pltpu.sync_copy(src_hbm.at[pl.ds(gid * ec, ec)], src_v)
        pltpu.sync_copy(dst_hbm.at[pl.ds(gid * ec, ec)], dst_v)
        plsc.subcore_barrier()
        pltpu.sync_copy(t_sh, t_local)

        def gather_body(i, _):
            j = i // 8
            q = pl.multiple_of((i % 8) * 16, 16)
            idx = src_v[j, pl.ds(q, 16)]
            vals_v[j, pl.ds(q, 16)] = plsc.load_gather(t_local, [idx])
            return 0
        lax.fori_loop(0, ec * 8, gather_body, 0)

        def fire(i, _):
            pltpu.async_copy(vals_v.at[i], acc_sh.at[dst_v.at[i]], sem, add=True)
            return 0
        lax.fori_loop(0, ec, fire, 0)

        def drain(i, _):
            pltpu.make_async_copy(vals_v.at[i], acc_sh.at[dst_v.at[i]], sem).wait()
            return 0
        lax.fori_loop(0, ec, drain, 0)
        plsc.subcore_barrier()
        pltpu.sync_copy(acc_sh.at[pl.ds(s * nslice, nslice)], buf_v)

        @pl.when(c == 0)
        def _():
            pltpu.sync_copy(buf_v, out_a.at[pl.ds(s * nslice, nslice)])

        @pl.when(c == 1)
        def _():
            pltpu.sync_copy(buf_v, out_b.at[pl.ds(s * nslice, nslice)])

    return deg_kernel, agg1_kernel, agg2_kernel


def kernel(obs, edge_index, W1, b1, W2, b2):
    n, fin = obs.shape
    fh = W1.shape[1]
    e = edge_index.shape[1]

    n_pad = ((n + NS * 128 - 1) // (NS * 128)) * (NS * 128)
    e_pad = ((e + NW * 1024 - 1) // (NW * 1024)) * (NW * 1024)
    ec = e_pad // NW // 128
    rows128 = n_pad // 128

    src = edge_index[0].astype(jnp.int32)
    dst = edge_index[1].astype(jnp.int32)
    src2d = jnp.concatenate(
        [src, jnp.zeros((e_pad - e,), jnp.int32)]).reshape(e_pad // 128, 128)
    dst2d = jnp.concatenate(
        [dst, jnp.full((e_pad - e,), n, jnp.int32)]).reshape(e_pad // 128, 128)
    obs_p = jnp.pad(obs, ((0, n_pad - n), (0, 0)))

    deg_kernel, agg1_kernel, agg2_kernel = _make_sc_kernels(n_pad, ec, fh)

    dega, degb = deg_kernel(dst2d)

    def _prep(obs_ref, w1_ref, dega_ref, degb_ref, u_ref, d_ref):
        deg = dega_ref[...] + degb_ref[...] + 1.0
        d = lax.rsqrt(deg)
        xw = jnp.dot(obs_ref[...].astype(jnp.bfloat16),
                     w1_ref[...].astype(jnp.bfloat16),
                     preferred_element_type=jnp.float32)
        u_ref[...] = jnp.reshape(xw, (rows128, 128, fh)) * d[:, :, None]
        d_ref[...] = d

    u3, dmat = pl.pallas_call(
        _prep,
        out_shape=(jax.ShapeDtypeStruct((rows128, 128, fh), jnp.float32),
                   jax.ShapeDtypeStruct((rows128, 128), jnp.float32)),
    )(obs_p, W1, dega.reshape(rows128, 128), degb.reshape(rows128, 128))

    acc1a, acc1b = agg1_kernel(u3.reshape(n_pad, fh), src2d, dst2d)

    def _mid(acca_ref, accb_ref, u_ref, d_ref, b1_ref, w2_ref, t_ref):
        d = d_ref[...]
        h1 = jnp.maximum(
            (acca_ref[...] + accb_ref[...] + u_ref[...]) * d[:, :, None]
            + b1_ref[...], 0.0)
        h1b = h1.astype(jnp.bfloat16).astype(jnp.float32)
        w2b = w2_ref[...].astype(jnp.bfloat16).astype(jnp.float32)
        t_ref[...] = jnp.sum(h1b * w2b, axis=2) * d

    t = pl.pallas_call(
        _mid,
        out_shape=jax.ShapeDtypeStruct((rows128, 128), jnp.float32),
    )(acc1a.reshape(rows128, 128, fh), acc1b.reshape(rows128, 128, fh),
      u3, dmat, b1.reshape(1, 1, fh), W2.reshape(1, 1, fh))

    acc2a, acc2b = agg2_kernel(t.reshape(n_pad), src2d, dst2d)

    def _fin(acca_ref, accb_ref, t_ref, d_ref, b2_ref, h2_ref):
        h2_ref[...] = jnp.maximum(
            (acca_ref[...] + accb_ref[...] + t_ref[...]) * d_ref[...]
            + b2_ref[...], 0.0)

    h2 = pl.pallas_call(
        _fin,
        out_shape=jax.ShapeDtypeStruct((rows128, 128), jnp.float32),
    )(acc2a.reshape(rows128, 128), acc2b.reshape(rows128, 128), t, dmat,
      b2.reshape(1, 1))

    return h2.reshape(n_pad)[:n].reshape(-1, 15)[:, 3:].reshape(-1)
